# baseline (speedup 1.0000x reference)
import functools

import jax
import jax.numpy as jnp
from jax import lax
from jax.experimental import pallas as pl
from jax.experimental.pallas import tpu as pltpu
from jax.experimental.pallas import tpu_sc as plsc

E_ = 8
T_ = 4096
DIN_ = 2048
DOUT_ = 2048

RB = 128
NRB = T_ // RB
S_MAX = NRB + E_ - 1

NW = 32
PW = T_ // NW
CH = 32
CH_N = PW // CH


def _routing_body(ids_ref, pos_ref, off_ref):
    ids = ids_ref[...]
    cu = (lax.broadcasted_iota(jnp.int32, (RB, RB), 0) <
          lax.broadcasted_iota(jnp.int32, (RB, RB), 1)).astype(jnp.float32)
    cl = (lax.broadcasted_iota(jnp.int32, (NRB, NRB), 1) <
          lax.broadcasted_iota(jnp.int32, (NRB, NRB), 0)).astype(jnp.float32)
    posf = jnp.zeros((NRB, RB), jnp.float32)
    offm = jnp.zeros((E_, 128), jnp.float32)
    rowi = lax.broadcasted_iota(jnp.int32, (E_, 128), 0)
    off = jnp.float32(0.0)
    for e in range(E_):
        m = (ids == e).astype(jnp.float32)
        p = jnp.dot(m, cu, preferred_element_type=jnp.float32)
        s = jnp.sum(m, axis=1, keepdims=True)
        q = jnp.dot(cl, s, preferred_element_type=jnp.float32)
        offm = jnp.where(rowi == e, off, offm)
        posf = posf + m * (off + q + p)
        off = off + jnp.sum(s)
    pos_ref[...] = posf.astype(jnp.int32)
    off_ref[...] = offm.astype(jnp.int32)


def _routing(ids2d):
    return pl.pallas_call(
        _routing_body,
        out_shape=(jax.ShapeDtypeStruct((NRB, RB), jnp.int32),
                   jax.ShapeDtypeStruct((E_, 128), jnp.int32)),
    )(ids2d)


def _schedule(offsets):
    starts = offsets
    ends = jnp.concatenate([offsets[1:], jnp.array([T_], jnp.int32)])
    r = jnp.arange(NRB, dtype=jnp.int32)[:, None]
    ov = (starts[None, :] < (r + 1) * RB) & (ends[None, :] > r * RB)
    flat = jnp.where(ov.reshape(-1),
                     jnp.arange(NRB * E_, dtype=jnp.int32),
                     jnp.int32(NRB * E_))
    order = jnp.sort(flat)[:S_MAX]
    nreal = jnp.sum(ov.astype(jnp.int32))
    last = order[nreal - 1]
    steps = jnp.where(jnp.arange(S_MAX) < nreal, order, last)
    step_r = steps // E_
    step_e = steps % E_
    lo = jnp.maximum(starts[step_e], step_r * RB) - step_r * RB
    hi = jnp.minimum(ends[step_e], (step_r + 1) * RB) - step_r * RB
    return step_r, step_e, lo, hi


def _sc_mesh():
    return plsc.VectorSubcoreMesh(core_axis_name="c", subcore_axis_name="s")


def _dispatch(tokens, pos3):
    @functools.partial(
        pl.kernel, mesh=_sc_mesh(),
        out_type=jax.ShapeDtypeStruct((T_, DIN_), jnp.float32),
        scratch_types=[
            pltpu.VMEM((CH_N, CH), jnp.int32),
            pltpu.VMEM((CH, DIN_), jnp.float32),
            pltpu.SemaphoreType.DMA,
        ],
    )
    def k(tok_hbm, pos_hbm, out_hbm, idx_v, rows_v, sem):
        wid = lax.axis_index("s") * 2 + lax.axis_index("c")
        base = wid * PW
        pltpu.sync_copy(pos_hbm.at[wid], idx_v)
        for c in range(CH_N):
            pltpu.sync_copy(tok_hbm.at[pl.ds(base + c * CH, CH)], rows_v)
            pltpu.async_copy(rows_v, out_hbm.at[idx_v.at[c]], sem).wait()

    return k(tokens, pos3)


def _combine(y_sorted, pos3):
    @functools.partial(
        pl.kernel, mesh=_sc_mesh(),
        out_type=jax.ShapeDtypeStruct((T_, DOUT_), jnp.float32),
        scratch_types=[
            pltpu.VMEM((CH_N, CH), jnp.int32),
            pltpu.VMEM((CH, DOUT_), jnp.float32),
            pltpu.SemaphoreType.DMA,
        ],
    )
    def k(ys_hbm, pos_hbm, out_hbm, idx_v, rows_v, sem):
        wid = lax.axis_index("s") * 2 + lax.axis_index("c")
        base = wid * PW
        pltpu.sync_copy(pos_hbm.at[wid], idx_v)
        for c in range(CH_N):
            pltpu.async_copy(ys_hbm.at[idx_v.at[c]], rows_v, sem).wait()
            pltpu.sync_copy(rows_v, out_hbm.at[pl.ds(base + c * CH, CH)])

    return k(y_sorted, pos3)


def _gmm_body(sr_ref, se_ref, lo_ref, hi_ref, x_ref, w_ref, o_ref):
    s = pl.program_id(0)
    acc = jnp.dot(x_ref[...].astype(jnp.bfloat16), w_ref[0],
                  preferred_element_type=jnp.float32)
    ri = lax.broadcasted_iota(jnp.int32, (RB, DOUT_), 0)
    msk = (ri >= lo_ref[s]) & (ri < hi_ref[s])
    o_ref[...] = jnp.where(msk, acc, o_ref[...])


def _gmm(step_r, step_e, lo, hi, x_sorted, weight):
    grid_spec = pltpu.PrefetchScalarGridSpec(
        num_scalar_prefetch=4,
        grid=(S_MAX,),
        in_specs=[
            pl.BlockSpec((RB, DIN_), lambda s, sr, se, lo, hi: (sr[s], 0)),
            pl.BlockSpec((1, DIN_, DOUT_), lambda s, sr, se, lo, hi: (se[s], 0, 0)),
        ],
        out_specs=pl.BlockSpec((RB, DOUT_), lambda s, sr, se, lo, hi: (sr[s], 0)),
    )
    return pl.pallas_call(
        _gmm_body,
        grid_spec=grid_spec,
        out_shape=jax.ShapeDtypeStruct((T_, DOUT_), jnp.float32),
    )(step_r, step_e, lo, hi, x_sorted, weight)


def kernel(tokens, exp_ids, weight):
    ids2d = exp_ids.reshape(NRB, RB)
    pos2d, offm = _routing(ids2d)
    offsets = offm[:, 0]
    step_r, step_e, lo, hi = _schedule(offsets)
    pos3 = pos2d.reshape(NW, CH_N, CH)
    x_sorted = _dispatch(tokens, pos3)
    y_sorted = _gmm(step_r, step_e, lo, hi, x_sorted,
                    weight.astype(jnp.bfloat16))
    return _combine(y_sorted, pos3)


# speedup vs baseline: 1.1985x; 1.1985x over previous
import functools

import jax
import jax.numpy as jnp
from jax import lax
from jax.experimental import pallas as pl
from jax.experimental.pallas import tpu as pltpu
from jax.experimental.pallas import tpu_sc as plsc

E_ = 8
T_ = 4096
DIN_ = 2048
DOUT_ = 2048

RB = 128
NRB = T_ // RB
S_MAX = NRB + E_ - 1

NW = 32
PW = T_ // NW
CH = 32
CH_N = PW // CH


def _routing_body(ids_ref, pos_ref, off_ref):
    ids = ids_ref[...]
    cu = (lax.broadcasted_iota(jnp.int32, (RB, RB), 0) <
          lax.broadcasted_iota(jnp.int32, (RB, RB), 1)).astype(jnp.float32)
    cl = (lax.broadcasted_iota(jnp.int32, (NRB, NRB), 1) <
          lax.broadcasted_iota(jnp.int32, (NRB, NRB), 0)).astype(jnp.float32)
    posf = jnp.zeros((NRB, RB), jnp.float32)
    offm = jnp.zeros((E_, 128), jnp.float32)
    rowi = lax.broadcasted_iota(jnp.int32, (E_, 128), 0)
    off = jnp.float32(0.0)
    for e in range(E_):
        m = (ids == e).astype(jnp.float32)
        p = jnp.dot(m, cu, preferred_element_type=jnp.float32)
        s = jnp.sum(m, axis=1, keepdims=True)
        q = jnp.dot(cl, s, preferred_element_type=jnp.float32)
        offm = jnp.where(rowi == e, off, offm)
        posf = posf + m * (off + q + p)
        off = off + jnp.sum(s)
    pos_ref[...] = posf.astype(jnp.int32)
    off_ref[...] = offm.astype(jnp.int32)


def _routing(ids2d):
    return pl.pallas_call(
        _routing_body,
        out_shape=(jax.ShapeDtypeStruct((NRB, RB), jnp.int32),
                   jax.ShapeDtypeStruct((E_, 128), jnp.int32)),
    )(ids2d)


def _schedule(offsets):
    starts = offsets
    ends = jnp.concatenate([offsets[1:], jnp.array([T_], jnp.int32)])
    r = jnp.arange(NRB, dtype=jnp.int32)[:, None]
    ov = (starts[None, :] < (r + 1) * RB) & (ends[None, :] > r * RB)
    flat = jnp.where(ov.reshape(-1),
                     jnp.arange(NRB * E_, dtype=jnp.int32),
                     jnp.int32(NRB * E_))
    order = jnp.sort(flat)[:S_MAX]
    nreal = jnp.sum(ov.astype(jnp.int32))
    last = order[nreal - 1]
    steps = jnp.where(jnp.arange(S_MAX) < nreal, order, last)
    step_r = steps // E_
    step_e = steps % E_
    lo = jnp.maximum(starts[step_e], step_r * RB) - step_r * RB
    hi = jnp.minimum(ends[step_e], (step_r + 1) * RB) - step_r * RB
    return step_r, step_e, lo, hi


def _sc_mesh():
    return plsc.VectorSubcoreMesh(core_axis_name="c", subcore_axis_name="s")


def _dispatch(tokens, pos3):
    @functools.partial(
        pl.kernel, mesh=_sc_mesh(),
        out_type=jax.ShapeDtypeStruct((T_, DIN_), jnp.float32),
        scratch_types=[
            pltpu.VMEM((CH_N, CH), jnp.int32),
            pltpu.VMEM((CH, DIN_), jnp.float32),
            pltpu.SemaphoreType.DMA,
        ],
    )
    def k(tok_hbm, pos_hbm, out_hbm, idx_v, rows_v, sem):
        wid = lax.axis_index("s") * 2 + lax.axis_index("c")
        base = wid * PW
        pltpu.sync_copy(pos_hbm.at[wid], idx_v)
        for c in range(CH_N):
            pltpu.sync_copy(tok_hbm.at[pl.ds(base + c * CH, CH)], rows_v)
            pltpu.async_copy(rows_v, out_hbm.at[idx_v.at[c]], sem).wait()

    return k(tokens, pos3)


def _combine(y_sorted, pos3):
    @functools.partial(
        pl.kernel, mesh=_sc_mesh(),
        out_type=jax.ShapeDtypeStruct((T_, DOUT_), jnp.float32),
        scratch_types=[
            pltpu.VMEM((CH_N, CH), jnp.int32),
            pltpu.VMEM((CH, DOUT_), jnp.float32),
            pltpu.SemaphoreType.DMA,
        ],
    )
    def k(ys_hbm, pos_hbm, out_hbm, idx_v, rows_v, sem):
        wid = lax.axis_index("s") * 2 + lax.axis_index("c")
        base = wid * PW
        pltpu.sync_copy(pos_hbm.at[wid], idx_v)
        for c in range(CH_N):
            pltpu.async_copy(ys_hbm.at[idx_v.at[c]], rows_v, sem).wait()
            pltpu.sync_copy(rows_v, out_hbm.at[pl.ds(base + c * CH, CH)])

    return k(y_sorted, pos3)


def _gmm_body(sr_ref, se_ref, lo_ref, hi_ref, x_ref, w_ref, o_ref,
              wbf_ref, laste_ref):
    s = pl.program_id(0)
    e = se_ref[s]

    @pl.when(jnp.logical_or(s == 0, e != laste_ref[0]))
    def _():
        wbf_ref[...] = w_ref[0].astype(jnp.bfloat16)
        laste_ref[0] = e

    acc = jnp.dot(x_ref[...].astype(jnp.bfloat16), wbf_ref[...],
                  preferred_element_type=jnp.float32)
    ri = lax.broadcasted_iota(jnp.int32, (RB, DOUT_), 0)
    msk = (ri >= lo_ref[s]) & (ri < hi_ref[s])
    o_ref[...] = jnp.where(msk, acc, o_ref[...])


def _gmm(step_r, step_e, lo, hi, x_sorted, weight):
    grid_spec = pltpu.PrefetchScalarGridSpec(
        num_scalar_prefetch=4,
        grid=(S_MAX,),
        in_specs=[
            pl.BlockSpec((RB, DIN_), lambda s, sr, se, lo, hi: (sr[s], 0)),
            pl.BlockSpec((1, DIN_, DOUT_), lambda s, sr, se, lo, hi: (se[s], 0, 0)),
        ],
        out_specs=pl.BlockSpec((RB, DOUT_), lambda s, sr, se, lo, hi: (sr[s], 0)),
        scratch_shapes=[
            pltpu.VMEM((DIN_, DOUT_), jnp.bfloat16),
            pltpu.SMEM((1,), jnp.int32),
        ],
    )
    return pl.pallas_call(
        _gmm_body,
        grid_spec=grid_spec,
        out_shape=jax.ShapeDtypeStruct((T_, DOUT_), jnp.float32),
    )(step_r, step_e, lo, hi, x_sorted, weight)


def kernel(tokens, exp_ids, weight):
    ids2d = exp_ids.reshape(NRB, RB)
    pos2d, offm = _routing(ids2d)
    offsets = offm[:, 0]
    step_r, step_e, lo, hi = _schedule(offsets)
    pos3 = pos2d.reshape(NW, CH_N, CH)
    x_sorted = _dispatch(tokens, pos3)
    y_sorted = _gmm(step_r, step_e, lo, hi, x_sorted, weight)
    return _combine(y_sorted, pos3)


# speedup vs baseline: 1.2436x; 1.0376x over previous
import functools

import jax
import jax.numpy as jnp
from jax import lax
from jax.experimental import pallas as pl
from jax.experimental.pallas import tpu as pltpu
from jax.experimental.pallas import tpu_sc as plsc

E_ = 8
T_ = 4096
DIN_ = 2048
DOUT_ = 2048

RB = 128
NRB = T_ // RB
S_MAX = NRB + E_ - 1

NW = 32
PW = T_ // NW
CH = 32
CH_N = PW // CH


def _routing_body(ids_ref, pos_ref, off_ref):
    ids = ids_ref[...]
    cu = (lax.broadcasted_iota(jnp.int32, (RB, RB), 0) <
          lax.broadcasted_iota(jnp.int32, (RB, RB), 1)).astype(jnp.float32)
    cl = (lax.broadcasted_iota(jnp.int32, (NRB, NRB), 1) <
          lax.broadcasted_iota(jnp.int32, (NRB, NRB), 0)).astype(jnp.float32)
    posf = jnp.zeros((NRB, RB), jnp.float32)
    offm = jnp.zeros((E_, 128), jnp.float32)
    rowi = lax.broadcasted_iota(jnp.int32, (E_, 128), 0)
    off = jnp.float32(0.0)
    for e in range(E_):
        m = (ids == e).astype(jnp.float32)
        p = jnp.dot(m, cu, preferred_element_type=jnp.float32)
        s = jnp.sum(m, axis=1, keepdims=True)
        q = jnp.dot(cl, s, preferred_element_type=jnp.float32)
        offm = jnp.where(rowi == e, off, offm)
        posf = posf + m * (off + q + p)
        off = off + jnp.sum(s)
    pos_ref[...] = posf.astype(jnp.int32)
    off_ref[...] = offm.astype(jnp.int32)


def _routing(ids2d):
    return pl.pallas_call(
        _routing_body,
        out_shape=(jax.ShapeDtypeStruct((NRB, RB), jnp.int32),
                   jax.ShapeDtypeStruct((E_, 128), jnp.int32)),
    )(ids2d)


def _schedule(offsets):
    starts = offsets
    ends = jnp.concatenate([offsets[1:], jnp.array([T_], jnp.int32)])
    r = jnp.arange(NRB, dtype=jnp.int32)[:, None]
    ov = (starts[None, :] < (r + 1) * RB) & (ends[None, :] > r * RB)
    flat = jnp.where(ov.reshape(-1),
                     jnp.arange(NRB * E_, dtype=jnp.int32),
                     jnp.int32(NRB * E_))
    order = jnp.sort(flat)[:S_MAX]
    nreal = jnp.sum(ov.astype(jnp.int32))
    last = order[nreal - 1]
    steps = jnp.where(jnp.arange(S_MAX) < nreal, order, last)
    step_r = steps // E_
    step_e = steps % E_
    lo = jnp.maximum(starts[step_e], step_r * RB) - step_r * RB
    hi = jnp.minimum(ends[step_e], (step_r + 1) * RB) - step_r * RB
    return step_r, step_e, lo, hi


def _sc_mesh():
    return plsc.VectorSubcoreMesh(core_axis_name="c", subcore_axis_name="s")


def _dispatch(tokens, pos3):
    @functools.partial(
        pl.kernel, mesh=_sc_mesh(),
        out_type=jax.ShapeDtypeStruct((T_, DIN_), jnp.float32),
        scratch_types=[
            pltpu.VMEM((CH_N, CH), jnp.int32),
            pltpu.VMEM((CH, DIN_), jnp.float32),
            pltpu.SemaphoreType.DMA,
        ],
    )
    def k(tok_hbm, pos_hbm, out_hbm, idx_v, rows_v, sem):
        wid = lax.axis_index("s") * 2 + lax.axis_index("c")
        base = wid * PW
        pltpu.sync_copy(pos_hbm.at[wid], idx_v)
        for c in range(CH_N):
            pltpu.sync_copy(tok_hbm.at[pl.ds(base + c * CH, CH)], rows_v)
            pltpu.async_copy(rows_v, out_hbm.at[idx_v.at[c]], sem).wait()

    return k(tokens, pos3)


def _combine(y_sorted, pos3):
    @functools.partial(
        pl.kernel, mesh=_sc_mesh(),
        out_type=jax.ShapeDtypeStruct((T_, DOUT_), jnp.float32),
        scratch_types=[
            pltpu.VMEM((CH_N, CH), jnp.int32),
            pltpu.VMEM((CH, DOUT_), jnp.float32),
            pltpu.SemaphoreType.DMA,
        ],
    )
    def k(ys_hbm, pos_hbm, out_hbm, idx_v, rows_v, sem):
        wid = lax.axis_index("s") * 2 + lax.axis_index("c")
        base = wid * PW
        pltpu.sync_copy(pos_hbm.at[wid], idx_v)
        for c in range(CH_N):
            pltpu.async_copy(ys_hbm.at[idx_v.at[c]], rows_v, sem).wait()
            pltpu.sync_copy(rows_v, out_hbm.at[pl.ds(base + c * CH, CH)])

    return k(y_sorted, pos3)


def _gmm_body(sr_ref, se_ref, lo_ref, hi_ref, x_ref, w0_ref, w1_ref, o_ref):
    s = pl.program_id(0)
    x = x_ref[...]
    h = DIN_ // 2
    acc = (jnp.dot(x[:, :h], w0_ref[0, 0], preferred_element_type=jnp.float32)
           + jnp.dot(x[:, h:], w1_ref[0, 0], preferred_element_type=jnp.float32))
    ri = lax.broadcasted_iota(jnp.int32, (RB, DOUT_), 0)
    msk = (ri >= lo_ref[s]) & (ri < hi_ref[s])
    o_ref[...] = jnp.where(msk, acc, o_ref[...])


def _gmm(step_r, step_e, lo, hi, x_sorted, weight):
    w4 = weight.reshape(E_, 2, DIN_ // 2, DOUT_)
    grid_spec = pltpu.PrefetchScalarGridSpec(
        num_scalar_prefetch=4,
        grid=(S_MAX,),
        in_specs=[
            pl.BlockSpec((RB, DIN_), lambda s, sr, se, lo, hi: (sr[s], 0)),
            pl.BlockSpec((1, 1, DIN_ // 2, DOUT_),
                         lambda s, sr, se, lo, hi: (se[s], 0, 0, 0)),
            pl.BlockSpec((1, 1, DIN_ // 2, DOUT_),
                         lambda s, sr, se, lo, hi: (se[s], 1, 0, 0)),
        ],
        out_specs=pl.BlockSpec((RB, DOUT_), lambda s, sr, se, lo, hi: (sr[s], 0)),
    )
    return pl.pallas_call(
        _gmm_body,
        grid_spec=grid_spec,
        out_shape=jax.ShapeDtypeStruct((T_, DOUT_), jnp.float32),
    )(step_r, step_e, lo, hi, x_sorted, w4, w4)


def kernel(tokens, exp_ids, weight):
    ids2d = exp_ids.reshape(NRB, RB)
    pos2d, offm = _routing(ids2d)
    offsets = offm[:, 0]
    step_r, step_e, lo, hi = _schedule(offsets)
    pos3 = pos2d.reshape(NW, CH_N, CH)
    x_sorted = _dispatch(tokens, pos3)
    y_sorted = _gmm(step_r, step_e, lo, hi, x_sorted, weight)
    return _combine(y_sorted, pos3)
